# baseline (speedup 1.0000x reference)
import functools

import jax
import jax.numpy as jnp
from jax import lax
from jax.experimental import pallas as pl
from jax.experimental.pallas import tpu as pltpu
from jax.experimental.pallas import tpu_sc as plsc

VOCAB = 100000
D = 64
BATCH = 1024
STATES = 20
ATOMS = 20
PER_ROW = ATOMS * 3
NC = 2
NS = 16
NW = NC * NS
B_PER_W = BATCH // NW
ACT_PER_W = B_PER_W * STATES
N_BLK = B_PER_W + ACT_PER_W
NBUF = 4
LANES = 16
NCH = D // LANES
J_GRP = 15
SCALE = 1.0 / ATOMS


@functools.partial(
    pl.kernel,
    mesh=plsc.VectorSubcoreMesh(core_axis_name="c", subcore_axis_name="s"),
    out_type=(jax.ShapeDtypeStruct((BATCH, D), jnp.float32),
              jax.ShapeDtypeStruct((BATCH * STATES, D), jnp.float32)),
    compiler_params=pltpu.CompilerParams(use_tc_tiling_on_sc=False),
    scratch_types=[
        pltpu.VMEM((N_BLK, PER_ROW), jnp.int32),
        [pltpu.VMEM((PER_ROW, D), jnp.float32) for _ in range(NBUF)],
        [pltpu.VMEM((1, D), jnp.float32) for _ in range(NBUF)],
        [pltpu.SemaphoreType.DMA for _ in range(NBUF)],
        [pltpu.SemaphoreType.DMA for _ in range(NBUF)],
    ],
)
def _pooled_lookup(obs_idx_hbm, act_idx_hbm, table_hbm, obs_out_hbm,
                   act_out_hbm, idx_all, rows_bufs, out_bufs, semg, semo):
    wid = lax.axis_index("s") * NC + lax.axis_index("c")
    b0 = wid * B_PER_W

    pltpu.sync_copy(obs_idx_hbm.at[pl.ds(b0, B_PER_W)],
                    idx_all.at[pl.ds(0, B_PER_W)])
    pltpu.sync_copy(act_idx_hbm.at[pl.ds(b0 * STATES, ACT_PER_W)],
                    idx_all.at[pl.ds(B_PER_W, ACT_PER_W)])

    def gather(i, rows_b, sem_b):
        return pltpu.make_async_copy(
            table_hbm.at[idx_all.at[i]], rows_b, sem_b)

    def start_out_store(i, out_b, sem_b):
        @pl.when(i < B_PER_W)
        def _():
            pltpu.make_async_copy(
                out_b, obs_out_hbm.at[pl.ds(b0 + i, 1)], sem_b).start()

        @pl.when(i >= B_PER_W)
        def _():
            row = b0 * STATES + (i - B_PER_W)
            pltpu.make_async_copy(
                out_b, act_out_hbm.at[pl.ds(row, 1)], sem_b).start()

    def wait_out_store(out_b, sem_b):
        pltpu.make_async_copy(
            out_b, obs_out_hbm.at[pl.ds(0, 1)], sem_b).wait()

    for b in range(NBUF):
        gather(b, rows_bufs[b], semg[b]).start()

    zeros = jnp.zeros((LANES,), jnp.float32)

    def body(p, carry):
        for b in range(NBUF):
            rows_b, out_b, semg_b, semo_b = (
                rows_bufs[b], out_bufs[b], semg[b], semo[b])
            i = NBUF * p + b
            gather(i, rows_b, semg_b).wait()

            def jbody(jj, accs):
                accs = list(accs)
                for u in range(J_GRP):
                    row = jj * J_GRP + u
                    for c in range(NCH):
                        accs[c] = accs[c] + rows_b[row, pl.ds(c * LANES,
                                                              LANES)]
                return tuple(accs)

            accs = lax.fori_loop(0, PER_ROW // J_GRP, jbody, (zeros,) * NCH)

            @pl.when(i + NBUF < N_BLK)
            def _():
                gather(i + NBUF, rows_b, semg_b).start()

            @pl.when(i >= NBUF)
            def _():
                wait_out_store(out_b, semo_b)

            for c in range(NCH):
                out_b[0, pl.ds(c * LANES, LANES)] = accs[c] * SCALE
            start_out_store(i, out_b, semo_b)
        return carry

    lax.fori_loop(0, N_BLK // NBUF, body, 0)
    for b in range(NBUF):
        wait_out_store(out_bufs[b], semo[b])


def kernel(sub_index, derived_sub_indices, action_mask, table):
    obs2 = sub_index.reshape(BATCH, PER_ROW)
    act2 = derived_sub_indices.reshape(BATCH * STATES, PER_ROW)
    obs, act = _pooled_lookup(obs2, act2, table)
    return (obs, act.reshape(BATCH, STATES, D), action_mask)


# speedup vs baseline: 1.1556x; 1.1556x over previous
import functools

import jax
import jax.numpy as jnp
from jax import lax
from jax.experimental import pallas as pl
from jax.experimental.pallas import tpu as pltpu
from jax.experimental.pallas import tpu_sc as plsc

VOCAB = 100000
D = 64
BATCH = 1024
STATES = 20
ATOMS = 20
PER_ROW = ATOMS * 3
ROWS = BATCH * (1 + STATES)
NC = 2
NS = 16
NW = NC * NS
ROWS_PER_W = ROWS // NW
R_BLK = 2
IDX_BLK = R_BLK * PER_ROW
N_BLK = ROWS_PER_W // R_BLK
NBUF = 4
LANES = 16
NCH = D // LANES
J_GRP = 15
SCALE = 1.0 / ATOMS


@functools.partial(
    pl.kernel,
    mesh=plsc.VectorSubcoreMesh(core_axis_name="c", subcore_axis_name="s"),
    out_type=(jax.ShapeDtypeStruct((BATCH, D), jnp.float32),
              jax.ShapeDtypeStruct((BATCH * STATES, D), jnp.float32)),
    compiler_params=pltpu.CompilerParams(use_tc_tiling_on_sc=False),
    scratch_types=[
        pltpu.VMEM((ROWS_PER_W * PER_ROW,), jnp.int32),
        [pltpu.VMEM((IDX_BLK, D), jnp.float32) for _ in range(NBUF)],
        [pltpu.VMEM((R_BLK, D), jnp.float32) for _ in range(NBUF)],
        [pltpu.SemaphoreType.DMA for _ in range(NBUF)],
        [pltpu.SemaphoreType.DMA for _ in range(NBUF)],
    ],
)
def _pooled_lookup(idx_hbm, table_hbm, obs_out_hbm, act_out_hbm, idx_all,
                   rows_bufs, out_bufs, semg, semo):
    wid = lax.axis_index("s") * NC + lax.axis_index("c")
    row_base = wid * ROWS_PER_W

    pltpu.sync_copy(
        idx_hbm.at[pl.ds(row_base * PER_ROW, ROWS_PER_W * PER_ROW)], idx_all)

    def gather(i, rows_b, sem_b):
        return pltpu.make_async_copy(
            table_hbm.at[idx_all.at[pl.ds(i * IDX_BLK, IDX_BLK)]],
            rows_b, sem_b)

    def start_out_store(i, out_b, sem_b):
        row0 = row_base + i * R_BLK

        @pl.when(row0 < BATCH)
        def _():
            pltpu.make_async_copy(
                out_b, obs_out_hbm.at[pl.ds(row0, R_BLK)], sem_b).start()

        @pl.when(row0 >= BATCH)
        def _():
            pltpu.make_async_copy(
                out_b, act_out_hbm.at[pl.ds(row0 - BATCH, R_BLK)],
                sem_b).start()

    def wait_out_store(out_b, sem_b):
        pltpu.make_async_copy(
            out_b, obs_out_hbm.at[pl.ds(0, R_BLK)], sem_b).wait()

    for b in range(NBUF):
        gather(b, rows_bufs[b], semg[b]).start()

    zeros = jnp.zeros((LANES,), jnp.float32)

    def body(p, carry):
        for b in range(NBUF):
            rows_b, out_b, semg_b, semo_b = (
                rows_bufs[b], out_bufs[b], semg[b], semo[b])
            i = NBUF * p + b
            gather(i, rows_b, semg_b).wait()

            def jbody(jj, accs):
                accs = list(accs)
                for u in range(J_GRP):
                    for r in range(R_BLK):
                        row = r * PER_ROW + jj * J_GRP + u
                        for c in range(NCH):
                            accs[r * NCH + c] = accs[r * NCH + c] + (
                                rows_b[row, pl.ds(c * LANES, LANES)])
                return tuple(accs)

            accs = lax.fori_loop(0, PER_ROW // J_GRP, jbody,
                                 (zeros,) * (R_BLK * NCH))

            @pl.when(i + NBUF < N_BLK)
            def _():
                gather(i + NBUF, rows_b, semg_b).start()

            @pl.when(i >= NBUF)
            def _():
                wait_out_store(out_b, semo_b)

            for r in range(R_BLK):
                for c in range(NCH):
                    out_b[r, pl.ds(c * LANES, LANES)] = (
                        accs[r * NCH + c] * SCALE)
            start_out_store(i, out_b, semo_b)
        return carry

    lax.fori_loop(0, N_BLK // NBUF, body, 0)
    for b in range(NBUF):
        wait_out_store(out_bufs[b], semo[b])


def kernel(sub_index, derived_sub_indices, action_mask, table):
    obs_idx = sub_index.reshape(BATCH, PER_ROW)
    act_idx = derived_sub_indices.reshape(BATCH * STATES, PER_ROW)
    flat_idx = jnp.concatenate([obs_idx, act_idx], axis=0).reshape(-1)
    obs, act = _pooled_lookup(flat_idx, table)
    return (obs, act.reshape(BATCH, STATES, D), action_mask)
